# baseline (speedup 1.0000x reference)
import functools

import jax
import jax.numpy as jnp
from jax import lax
from jax.experimental import pallas as pl
from jax.experimental.pallas import tpu as pltpu
from jax.experimental.pallas import tpu_sc as plsc

_D = 128
_NW = 32
_CHUNK = 128
_S = 50
_NCH = 45
_NBUF = 5
_GLEAD = 3
_WLAG = _NBUF - _GLEAD


def _sc_gather(idx_t, weight, n_batch):
    mesh = plsc.VectorSubcoreMesh(core_axis_name="c", subcore_axis_name="s")

    @functools.partial(
        pl.kernel,
        out_type=jax.ShapeDtypeStruct((_S, n_batch, _D), jnp.float32),
        mesh=mesh,
        scratch_types=(
            [pltpu.VMEM((_S, _CHUNK), jnp.int32),
             pltpu.VMEM((_NBUF, _CHUNK, _D), jnp.float32)]
            + [pltpu.SemaphoreType.DMA] * (2 * _NBUF)
        ),
    )
    def k(idx_hbm, w_hbm, out_hbm, idx_v, rows_v, *sems):
        gsem, wsem = sems[:_NBUF], sems[_NBUF:]
        wid = lax.axis_index("s") * 2 + lax.axis_index("c")
        col0 = wid * _CHUNK
        pltpu.sync_copy(idx_hbm.at[:, pl.ds(col0, _CHUNK)], idx_v)

        def gather_start(c, b):
            pltpu.async_copy(w_hbm.at[idx_v.at[c]], rows_v.at[b], gsem[b])

        def gather_wait(b):
            pltpu.make_async_copy(
                w_hbm.at[idx_v.at[0]], rows_v.at[b], gsem[b]).wait()

        def write_start(c, b):
            pltpu.async_copy(
                rows_v.at[b], out_hbm.at[c, pl.ds(col0, _CHUNK)], wsem[b])

        def write_wait(b):
            pltpu.make_async_copy(
                rows_v.at[b], out_hbm.at[0, pl.ds(col0, _CHUNK)],
                wsem[b]).wait()

        def step(c, b, do_wwait, do_gstart):
            gather_wait(b)
            write_start(c, b)
            if do_wwait:
                write_wait((b - _WLAG) % _NBUF)
            if do_gstart:
                gather_start(c + _GLEAD, (b + _GLEAD) % _NBUF)

        for c in range(_GLEAD):
            gather_start(c, c % _NBUF)
        for c in range(_WLAG):
            step(c, c % _NBUF, do_wwait=False, do_gstart=True)

        @pl.loop(_WLAG, _NCH - _GLEAD, step=_NBUF)
        def _grp(g):
            for j in range(_NBUF):
                step(g + j, (_WLAG + j) % _NBUF, do_wwait=True, do_gstart=True)

        for c in range(_NCH - _GLEAD, _NCH):
            step(c, c % _NBUF, do_wwait=True, do_gstart=False)
        for c in range(_NCH - _WLAG, _NCH):
            write_wait(c % _NBUF)

    return k(idx_t, weight)


def kernel(token_ids, weight):
    n_batch, s = token_ids.shape
    idx_t = token_ids.T.astype(jnp.int32)
    out = _sc_gather(idx_t, weight, n_batch)
    tc_part = weight[idx_t[_NCH:, :]]
    out = lax.dynamic_update_slice(out, tc_part, (_NCH, 0, 0))
    return jnp.transpose(out, (1, 0, 2))


# speedup vs baseline: 1.1807x; 1.1807x over previous
import functools

import jax
import jax.numpy as jnp
from jax import lax
from jax.experimental import pallas as pl
from jax.experimental.pallas import tpu as pltpu
from jax.experimental.pallas import tpu_sc as plsc

_D = 128
_NW = 32
_CHUNK = 128
_NCH = 50
_NBUF = 5
_GLEAD = 3
_WLAG = _NBUF - _GLEAD


def _sc_gather(idx_t, weight, n_batch):
    mesh = plsc.VectorSubcoreMesh(core_axis_name="c", subcore_axis_name="s")

    @functools.partial(
        pl.kernel,
        out_type=jax.ShapeDtypeStruct((_NCH, n_batch, _D), jnp.float32),
        mesh=mesh,
        scratch_types=(
            [pltpu.VMEM((_NCH, _CHUNK), jnp.int32),
             pltpu.VMEM((_NBUF, _CHUNK, _D), jnp.float32)]
            + [pltpu.SemaphoreType.DMA] * (2 * _NBUF)
        ),
    )
    def k(idx_hbm, w_hbm, out_hbm, idx_v, rows_v, *sems):
        gsem, wsem = sems[:_NBUF], sems[_NBUF:]
        wid = lax.axis_index("s") * 2 + lax.axis_index("c")
        col0 = wid * _CHUNK
        pltpu.sync_copy(idx_hbm.at[:, pl.ds(col0, _CHUNK)], idx_v)

        def gather_start(c, b):
            pltpu.async_copy(w_hbm.at[idx_v.at[c]], rows_v.at[b], gsem[b])

        def gather_wait(b):
            pltpu.make_async_copy(
                w_hbm.at[idx_v.at[0]], rows_v.at[b], gsem[b]).wait()

        def write_start(c, b):
            pltpu.async_copy(
                rows_v.at[b], out_hbm.at[c, pl.ds(col0, _CHUNK)], wsem[b])

        def write_wait(b):
            pltpu.make_async_copy(
                rows_v.at[b], out_hbm.at[0, pl.ds(col0, _CHUNK)],
                wsem[b]).wait()

        def step(c, b, do_wwait, do_gstart):
            gather_wait(b)
            write_start(c, b)
            if do_wwait:
                write_wait((b - _WLAG) % _NBUF)
            if do_gstart:
                gather_start(c + _GLEAD, (b + _GLEAD) % _NBUF)

        for c in range(_GLEAD):
            gather_start(c, c % _NBUF)
        for c in range(_WLAG):
            step(c, c % _NBUF, do_wwait=False, do_gstart=True)

        @pl.loop(_WLAG, _NCH - _GLEAD, step=_NBUF)
        def _grp(g):
            for j in range(_NBUF):
                step(g + j, (_WLAG + j) % _NBUF, do_wwait=True, do_gstart=True)

        for c in range(_NCH - _GLEAD, _NCH):
            step(c, c % _NBUF, do_wwait=True, do_gstart=False)
        for c in range(_NCH - _WLAG, _NCH):
            write_wait(c % _NBUF)

    return k(idx_t, weight)


def kernel(token_ids, weight):
    n_batch, s = token_ids.shape
    idx_t = token_ids.T.astype(jnp.int32)
    out = _sc_gather(idx_t, weight, n_batch)
    return jnp.transpose(out, (1, 0, 2))


# speedup vs baseline: 1.1865x; 1.0050x over previous
import functools

import jax
import jax.numpy as jnp
from jax import lax
from jax.experimental import pallas as pl
from jax.experimental.pallas import tpu as pltpu
from jax.experimental.pallas import tpu_sc as plsc

_D = 128
_NW = 32
_COLS = 128
_CHUNK = 64
_NCH = 100
_NBUF = 10
_GLEAD = 6
_WLAG = _NBUF - _GLEAD


def _sc_gather(idx_t, weight, n_batch):
    mesh = plsc.VectorSubcoreMesh(core_axis_name="c", subcore_axis_name="s")

    @functools.partial(
        pl.kernel,
        out_type=jax.ShapeDtypeStruct((_NCH // 2, n_batch, _D), jnp.float32),
        mesh=mesh,
        scratch_types=(
            [pltpu.VMEM((_NCH // 2, _COLS), jnp.int32),
             pltpu.VMEM((_NBUF, _CHUNK, _D), jnp.float32)]
            + [pltpu.SemaphoreType.DMA] * (2 * _NBUF)
        ),
    )
    def k(idx_hbm, w_hbm, out_hbm, idx_v, rows_v, *sems):
        gsem, wsem = sems[:_NBUF], sems[_NBUF:]
        wid = lax.axis_index("s") * 2 + lax.axis_index("c")
        col0 = wid * _COLS
        pltpu.sync_copy(idx_hbm.at[:, pl.ds(col0, _COLS)], idx_v)

        def gather_start(c, b):
            s, h = c // 2, (c % 2) * _CHUNK
            pltpu.async_copy(
                w_hbm.at[idx_v.at[s, pl.ds(h, _CHUNK)]], rows_v.at[b],
                gsem[b])

        def gather_wait(b):
            pltpu.make_async_copy(
                w_hbm.at[idx_v.at[0, pl.ds(0, _CHUNK)]], rows_v.at[b],
                gsem[b]).wait()

        def write_start(c, b):
            s, h = c // 2, (c % 2) * _CHUNK
            pltpu.async_copy(
                rows_v.at[b], out_hbm.at[s, pl.ds(col0 + h, _CHUNK)], wsem[b])

        def write_wait(b):
            pltpu.make_async_copy(
                rows_v.at[b], out_hbm.at[0, pl.ds(col0, _CHUNK)],
                wsem[b]).wait()

        def step(c, b, do_wwait, do_gstart):
            gather_wait(b)
            write_start(c, b)
            if do_wwait:
                write_wait((b - _WLAG) % _NBUF)
            if do_gstart:
                gather_start(c + _GLEAD, (b + _GLEAD) % _NBUF)

        for c in range(_GLEAD):
            gather_start(c, c % _NBUF)
        for c in range(_WLAG):
            step(c, c % _NBUF, do_wwait=False, do_gstart=True)

        @pl.loop(_WLAG, _NCH - _GLEAD, step=_NBUF)
        def _grp(g):
            for j in range(_NBUF):
                step(g + j, (_WLAG + j) % _NBUF, do_wwait=True, do_gstart=True)

        for c in range(_NCH - _GLEAD, _NCH):
            step(c, c % _NBUF, do_wwait=True, do_gstart=False)
        for c in range(_NCH - _WLAG, _NCH):
            write_wait(c % _NBUF)

    return k(idx_t, weight)


def kernel(token_ids, weight):
    n_batch, s = token_ids.shape
    idx_t = token_ids.T.astype(jnp.int32)
    out = _sc_gather(idx_t, weight, n_batch)
    return jnp.transpose(out, (1, 0, 2))


# speedup vs baseline: 1.1867x; 1.0001x over previous
import functools

import jax
import jax.numpy as jnp
from jax import lax
from jax.experimental import pallas as pl
from jax.experimental.pallas import tpu as pltpu
from jax.experimental.pallas import tpu_sc as plsc

_D = 128
_NW = 32
_COLS = 128
_CHUNK = 64
_NCH = 100
_NBUF = 10
_GLEAD = 7
_WLAG = _NBUF - _GLEAD


def _sc_gather(idx_t, weight, n_batch):
    mesh = plsc.VectorSubcoreMesh(core_axis_name="c", subcore_axis_name="s")

    @functools.partial(
        pl.kernel,
        out_type=jax.ShapeDtypeStruct((_NCH // 2, n_batch, _D), jnp.float32),
        mesh=mesh,
        scratch_types=(
            [pltpu.VMEM((_NCH // 2, _COLS), jnp.int32),
             pltpu.VMEM((_NBUF, _CHUNK, _D), jnp.float32)]
            + [pltpu.SemaphoreType.DMA] * (2 * _NBUF)
        ),
    )
    def k(idx_hbm, w_hbm, out_hbm, idx_v, rows_v, *sems):
        gsem, wsem = sems[:_NBUF], sems[_NBUF:]
        wid = lax.axis_index("s") * 2 + lax.axis_index("c")
        col0 = wid * _COLS
        pltpu.sync_copy(idx_hbm.at[:, pl.ds(col0, _COLS)], idx_v)

        def gather_start(c, b):
            s, h = c // 2, (c % 2) * _CHUNK
            pltpu.async_copy(
                w_hbm.at[idx_v.at[s, pl.ds(h, _CHUNK)]], rows_v.at[b],
                gsem[b])

        def gather_wait(b):
            pltpu.make_async_copy(
                w_hbm.at[idx_v.at[0, pl.ds(0, _CHUNK)]], rows_v.at[b],
                gsem[b]).wait()

        def write_start(c, b):
            s, h = c // 2, (c % 2) * _CHUNK
            pltpu.async_copy(
                rows_v.at[b], out_hbm.at[s, pl.ds(col0 + h, _CHUNK)], wsem[b])

        def write_wait(b):
            pltpu.make_async_copy(
                rows_v.at[b], out_hbm.at[0, pl.ds(col0, _CHUNK)],
                wsem[b]).wait()

        def step(c, b, do_wwait, do_gstart):
            gather_wait(b)
            write_start(c, b)
            if do_wwait:
                write_wait((b - _WLAG) % _NBUF)
            if do_gstart:
                gather_start(c + _GLEAD, (b + _GLEAD) % _NBUF)

        for c in range(_GLEAD):
            gather_start(c, c % _NBUF)
        for c in range(_WLAG):
            step(c, c % _NBUF, do_wwait=False, do_gstart=True)

        @pl.loop(_WLAG, _NCH - _GLEAD, step=_NBUF)
        def _grp(g):
            for j in range(_NBUF):
                step(g + j, (_WLAG + j) % _NBUF, do_wwait=True, do_gstart=True)

        for c in range(_NCH - _GLEAD, _NCH):
            step(c, c % _NBUF, do_wwait=True, do_gstart=False)
        for c in range(_NCH - _WLAG, _NCH):
            write_wait(c % _NBUF)

    return k(idx_t, weight)


def kernel(token_ids, weight):
    n_batch, s = token_ids.shape
    idx_t = token_ids.T.astype(jnp.int32)
    out = _sc_gather(idx_t, weight, n_batch)
    return jnp.transpose(out, (1, 0, 2))
